# Initial kernel scaffold; baseline (speedup 1.0000x reference)
import jax
import jax.numpy as jnp
from jax.experimental import pallas as pl


def kernel(x, bboxes):
    raise NotImplementedError("write your pallas kernel here")


# speedup vs baseline: 8.2367x; 8.2367x over previous
import functools

import jax
import jax.numpy as jnp
from jax.experimental import pallas as pl
from jax.experimental.pallas import tpu as pltpu

B, C, Hf, Wf = 4, 768, 32, 32
NB = 32
OUT = 7
SR = 2
SCALE = 1.0 / 16.0
HW = Hf * Wf
RCOLS = OUT * OUT


def _axis_weights(lo, hi, extent):
    c1 = lo * SCALE - 0.5
    c2 = hi * SCALE - 0.5
    binsz = (c2 - c1) / OUT
    pix = jax.lax.broadcasted_iota(jnp.int32, (NB, extent, OUT), 1).astype(jnp.float32)
    ob = jax.lax.broadcasted_iota(jnp.int32, (NB, extent, OUT), 2).astype(jnp.float32)
    acc = jnp.zeros((NB, extent, OUT), jnp.float32)
    for k in range(SR):
        off = (SR * ob + k + 0.5) / SR
        s = c1 + off * binsz
        valid = jnp.where((s >= -1.0) & (s <= float(extent)), 1.0, 0.0)
        sc = jnp.clip(s, 0.0, float(extent - 1))
        i0 = jnp.floor(sc)
        frac = sc - i0
        i1 = jnp.minimum(i0 + 1.0, float(extent - 1))
        w = jnp.where(pix == i0, 1.0 - frac, 0.0) + jnp.where(pix == i1, frac, 0.0)
        acc = acc + w * valid
    return acc * (1.0 / SR)


def _roi_kernel(bb_ref, x_ref, out_ref, mt_ref):
    bb = bb_ref[0]
    ayT = _axis_weights(bb[:, 1:2, None], bb[:, 3:4, None], Hf)
    axT = _axis_weights(bb[:, 0:1, None], bb[:, 2:3, None], Wf)
    for r in range(NB):
        aye = jnp.broadcast_to(ayT[r][:, None, :], (Hf, Wf, OUT)).reshape(HW, OUT)
        axe = jnp.broadcast_to(axT[r][None, :, :], (Hf, Wf, OUT)).reshape(HW, OUT)
        mt = jnp.concatenate([aye[:, o : o + 1] * axe for o in range(OUT)], axis=1)
        mt_ref[:, r * RCOLS : (r + 1) * RCOLS] = mt
    big = jnp.dot(
        x_ref[0], mt_ref[...], preferred_element_type=jnp.float32,
        precision=jax.lax.Precision.HIGHEST,
    )
    for r in range(NB):
        out_ref[0, r] = big[:, r * RCOLS : (r + 1) * RCOLS]


@jax.jit
def kernel(x, bboxes):
    xf = x.reshape(B, C, HW)
    out = pl.pallas_call(
        _roi_kernel,
        grid=(B,),
        in_specs=[
            pl.BlockSpec((1, NB, 4), lambda b: (b, 0, 0)),
            pl.BlockSpec((1, C, HW), lambda b: (b, 0, 0)),
        ],
        out_specs=pl.BlockSpec((1, NB, C, RCOLS), lambda b: (b, 0, 0, 0)),
        out_shape=jax.ShapeDtypeStruct((B, NB, C, RCOLS), jnp.float32),
        scratch_shapes=[pltpu.VMEM((HW, NB * RCOLS), jnp.float32)],
    )(bboxes, xf)
    return out.reshape(B * NB, C, OUT, OUT)


# speedup vs baseline: 8.4187x; 1.0221x over previous
import functools

import jax
import jax.numpy as jnp
from jax.experimental import pallas as pl
from jax.experimental.pallas import tpu as pltpu

B, C, Hf, Wf = 4, 768, 32, 32
NB = 32
OUT = 7
SR = 2
SCALE = 1.0 / 16.0
HW = Hf * Wf
RCOLS = OUT * OUT


def _axis_weights(lo, hi, extent):
    c1 = lo * SCALE - 0.5
    c2 = hi * SCALE - 0.5
    binsz = (c2 - c1) / OUT
    pix = jax.lax.broadcasted_iota(jnp.int32, (NB, extent, OUT), 1).astype(jnp.float32)
    ob = jax.lax.broadcasted_iota(jnp.int32, (NB, extent, OUT), 2).astype(jnp.float32)
    acc = jnp.zeros((NB, extent, OUT), jnp.float32)
    for k in range(SR):
        off = (SR * ob + k + 0.5) / SR
        s = c1 + off * binsz
        valid = jnp.where((s >= -1.0) & (s <= float(extent)), 1.0, 0.0)
        sc = jnp.clip(s, 0.0, float(extent - 1))
        i0 = jnp.floor(sc)
        frac = sc - i0
        i1 = jnp.minimum(i0 + 1.0, float(extent - 1))
        w = jnp.where(pix == i0, 1.0 - frac, 0.0) + jnp.where(pix == i1, frac, 0.0)
        acc = acc + w * valid
    return acc * (1.0 / SR)


def _roi_kernel(bb_ref, x_ref, out_ref, mt_ref):
    bb = bb_ref[0]
    ayT = _axis_weights(bb[:, 1:2, None], bb[:, 3:4, None], Hf)
    axT = _axis_weights(bb[:, 0:1, None], bb[:, 2:3, None], Wf)
    for r in range(NB):
        aye = jnp.broadcast_to(ayT[r][:, None, :], (Hf, Wf, OUT)).reshape(HW, OUT)
        axe = jnp.broadcast_to(axT[r][None, :, :], (Hf, Wf, OUT)).reshape(HW, OUT)
        mt = jnp.concatenate([aye[:, o : o + 1] * axe for o in range(OUT)], axis=1)
        mt_ref[:, r * RCOLS : (r + 1) * RCOLS] = mt
    big = jnp.dot(
        x_ref[0], mt_ref[...], preferred_element_type=jnp.float32,
    )
    for r in range(NB):
        out_ref[0, r] = big[:, r * RCOLS : (r + 1) * RCOLS]


@jax.jit
def kernel(x, bboxes):
    xf = x.reshape(B, C, HW)
    out = pl.pallas_call(
        _roi_kernel,
        grid=(B,),
        in_specs=[
            pl.BlockSpec((1, NB, 4), lambda b: (b, 0, 0)),
            pl.BlockSpec((1, C, HW), lambda b: (b, 0, 0)),
        ],
        out_specs=pl.BlockSpec((1, NB, C, RCOLS), lambda b: (b, 0, 0, 0)),
        out_shape=jax.ShapeDtypeStruct((B, NB, C, RCOLS), jnp.float32),
        scratch_shapes=[pltpu.VMEM((HW, NB * RCOLS), jnp.float32)],
    )(bboxes, xf)
    return out.reshape(B * NB, C, OUT, OUT)


# speedup vs baseline: 33.8717x; 4.0234x over previous
import jax
import jax.numpy as jnp
from jax.experimental import pallas as pl

B, C, Hf, Wf = 4, 768, 32, 32
NB = 32
OUT = 7
SR = 2
SCALE = 1.0 / 16.0
HW = Hf * Wf
RCOLS = OUT * OUT
NCOL = NB * RCOLS


def _fdiv(a, d):
    return jnp.floor((a + 0.5) * (1.0 / d))


def _axis_factor(shape_dim, c1, binsz, o_row, extent):
    pix = jax.lax.broadcasted_iota(jnp.int32, (extent, NCOL), 0).astype(jnp.float32)
    acc = jnp.zeros((extent, NCOL), jnp.float32)
    for k in range(SR):
        off = (SR * o_row + k + 0.5) * (1.0 / SR)
        s = c1 + off * binsz
        valid = jnp.where((s >= -1.0) & (s <= float(extent)), 0.5, 0.0)
        sc = jnp.clip(s, 0.0, float(extent - 1))
        i0 = jnp.floor(sc)
        frac = sc - i0
        i1 = jnp.minimum(i0 + 1.0, float(extent - 1))
        hi = (1.0 - frac) * valid
        lo = frac * valid
        acc = acc + jnp.where(pix == i0, hi, 0.0) + jnp.where(pix == i1, lo, 0.0)
    return acc


def _roi_kernel(bb_ref, x_ref, out_ref):
    bbT = bb_ref[0]
    x1 = bbT[0:1, :] * SCALE - 0.5
    y1 = bbT[1:2, :] * SCALE - 0.5
    x2 = bbT[2:3, :] * SCALE - 0.5
    y2 = bbT[3:4, :] * SCALE - 0.5
    params = jnp.concatenate(
        [y1, (y2 - y1) * (1.0 / OUT), x1, (x2 - x1) * (1.0 / OUT)], axis=0
    )

    jcol = jax.lax.broadcasted_iota(jnp.int32, (1, NCOL), 1).astype(jnp.float32)
    rrow = jax.lax.broadcasted_iota(jnp.int32, (NB, NCOL), 0).astype(jnp.float32)
    rloc = _fdiv(jcol, RCOLS)
    onehotT = jnp.where(rrow == rloc, 1.0, 0.0)
    prow = jnp.dot(
        params, onehotT, preferred_element_type=jnp.float32,
        precision=jax.lax.Precision.HIGHEST,
    )

    jin = jcol - RCOLS * rloc
    oy = _fdiv(jin, OUT)
    ox = jin - OUT * oy

    cy = _axis_factor(Hf, prow[0:1, :], prow[1:2, :], oy, Hf)
    cx = _axis_factor(Wf, prow[2:3, :], prow[3:4, :], ox, Wf)
    mt = (cy[:, None, :] * cx[None, :, :]).reshape(HW, NCOL)

    big = jnp.dot(x_ref[0], mt, preferred_element_type=jnp.float32)
    for r in range(NB):
        out_ref[0, r] = big[:, r * RCOLS : (r + 1) * RCOLS]


@jax.jit
def kernel(x, bboxes):
    xf = x.reshape(B, C, HW)
    bbT = bboxes.transpose(0, 2, 1)
    out = pl.pallas_call(
        _roi_kernel,
        grid=(B,),
        in_specs=[
            pl.BlockSpec((1, 4, NB), lambda b: (b, 0, 0)),
            pl.BlockSpec((1, C, HW), lambda b: (b, 0, 0)),
        ],
        out_specs=pl.BlockSpec((1, NB, C, RCOLS), lambda b: (b, 0, 0, 0)),
        out_shape=jax.ShapeDtypeStruct((B, NB, C, RCOLS), jnp.float32),
    )(bbT, xf)
    return out.reshape(B * NB, C, OUT, OUT)
